# baseline (speedup 1.0000x reference)
import functools

import jax
import jax.numpy as jnp
from jax import lax
from jax.experimental import pallas as pl
from jax.experimental.pallas import tpu as pltpu
from jax.experimental.pallas import tpu_sc as plsc

DIM = 32
OUT_LANES = 128
NUM_WORKERS = 32
CHUNK = 640
NBUF = 4


@functools.lru_cache(maxsize=None)
def _make_gather(total_rows: int):
    rows_per_worker = total_rows // NUM_WORKERS
    nchunks = rows_per_worker // CHUNK
    nblocks = nchunks // NBUF
    assert rows_per_worker % CHUNK == 0 and nchunks % NBUF == 0 and nchunks >= 4
    mesh = plsc.VectorSubcoreMesh(core_axis_name="c", subcore_axis_name="s")

    @functools.partial(
        pl.kernel,
        mesh=mesh,
        compiler_params=pltpu.CompilerParams(
            use_tc_tiling_on_sc=False, skip_device_barrier=True
        ),
        out_type=jax.ShapeDtypeStruct((total_rows, OUT_LANES), jnp.float32),
        scratch_types=[
            pltpu.VMEM((rows_per_worker,), jnp.int32),
            pltpu.VMEM((NBUF, CHUNK, DIM), jnp.float32),
            [pltpu.SemaphoreType.DMA] * NBUF,
            [pltpu.SemaphoreType.DMA] * NBUF,
        ],
    )
    def gather_kernel(idx_hbm, table_hbm, out_hbm, idx_v, rows_v, sem_g, sem_s):
        wid = lax.axis_index("s") * 2 + lax.axis_index("c")
        base = wid * rows_per_worker

        pltpu.sync_copy(idx_hbm.at[pl.ds(base, rows_per_worker)], idx_v)

        def start_gather(g, b):
            pltpu.async_copy(
                table_hbm.at[idx_v.at[pl.ds(g * CHUNK, CHUNK)]],
                rows_v.at[b],
                sem_g[b],
            )

        def wait_gather(b):
            pltpu.make_async_copy(
                table_hbm.at[idx_v.at[pl.ds(0, CHUNK)]], rows_v.at[b], sem_g[b]
            ).wait()

        def out_slice(g):
            return out_hbm.at[
                pl.ds(base + g * CHUNK, CHUNK), pl.ds(0, DIM)
            ]

        def start_store(g, b):
            pltpu.async_copy(rows_v.at[b], out_slice(g), sem_s[b])

        def wait_store(b):
            pltpu.make_async_copy(rows_v.at[b], out_slice(0), sem_s[b]).wait()

        start_gather(0, 0)
        start_gather(1, 1)

        def block(blk, carry):
            for b in range(NBUF):
                g = blk * NBUF + b
                wait_gather(b)
                nb = (b + 2) % NBUF

                @pl.when(g >= 2)
                def _():
                    wait_store(nb)

                @pl.when(g + 2 < nchunks)
                def _():
                    start_gather(g + 2, nb)

                start_store(g, b)
            return carry

        lax.fori_loop(0, nblocks, block, 0)

        wait_store((nchunks - 2) % NBUF)
        wait_store((nchunks - 1) % NBUF)

    return gather_kernel


@jax.jit
def kernel(inputs, table):
    b, t = inputs.shape
    idx = inputs.reshape(b * t).astype(jnp.int32)
    out = _make_gather(b * t)(idx, table)
    return out[:, :DIM].reshape(b, t, DIM)


# speedup vs baseline: 1.0002x; 1.0002x over previous
import functools

import jax
import jax.numpy as jnp
from jax import lax
from jax.experimental import pallas as pl
from jax.experimental.pallas import tpu as pltpu
from jax.experimental.pallas import tpu_sc as plsc

DIM = 32
OUT_LANES = 128
NUM_WORKERS = 32
CHUNK = 640
NBUF = 4


@functools.lru_cache(maxsize=None)
def _make_gather(total_rows: int):
    rows_per_worker = total_rows // NUM_WORKERS
    nchunks = rows_per_worker // CHUNK
    nblocks = nchunks // NBUF
    assert rows_per_worker % CHUNK == 0 and nchunks % NBUF == 0 and nchunks >= 4
    mesh = plsc.VectorSubcoreMesh(core_axis_name="c", subcore_axis_name="s")

    @functools.partial(
        pl.kernel,
        mesh=mesh,
        compiler_params=pltpu.CompilerParams(use_tc_tiling_on_sc=False),
        out_type=jax.ShapeDtypeStruct((total_rows, OUT_LANES), jnp.float32),
        scratch_types=[
            pltpu.VMEM((rows_per_worker,), jnp.int32),
            pltpu.VMEM((NBUF, CHUNK, DIM), jnp.float32),
            [pltpu.SemaphoreType.DMA] * NBUF,
            [pltpu.SemaphoreType.DMA] * NBUF,
        ],
    )
    def gather_kernel(idx_hbm, table_hbm, out_hbm, idx_v, rows_v, sem_g, sem_s):
        wid = lax.axis_index("s") * 2 + lax.axis_index("c")
        base = wid * rows_per_worker

        pltpu.sync_copy(idx_hbm.at[pl.ds(base, rows_per_worker)], idx_v)

        def start_gather(g, b):
            pltpu.async_copy(
                table_hbm.at[idx_v.at[pl.ds(g * CHUNK, CHUNK)]],
                rows_v.at[b],
                sem_g[b],
            )

        def wait_gather(b):
            pltpu.make_async_copy(
                table_hbm.at[idx_v.at[pl.ds(0, CHUNK)]], rows_v.at[b], sem_g[b]
            ).wait()

        def out_slice(g):
            return out_hbm.at[
                pl.ds(base + g * CHUNK, CHUNK), pl.ds(0, DIM)
            ]

        def start_store(g, b):
            pltpu.async_copy(rows_v.at[b], out_slice(g), sem_s[b])

        def wait_store(b):
            pltpu.make_async_copy(rows_v.at[b], out_slice(0), sem_s[b]).wait()

        start_gather(0, 0)
        start_gather(1, 1)

        def block(blk, carry):
            for b in range(NBUF):
                g = blk * NBUF + b
                wait_gather(b)
                nb = (b + 2) % NBUF

                @pl.when(g >= 2)
                def _():
                    wait_store(nb)

                @pl.when(g + 2 < nchunks)
                def _():
                    start_gather(g + 2, nb)

                start_store(g, b)
            return carry

        lax.fori_loop(0, nblocks, block, 0)

        wait_store((nchunks - 2) % NBUF)
        wait_store((nchunks - 1) % NBUF)

    return gather_kernel


@jax.jit
def kernel(inputs, table):
    b, t = inputs.shape
    idx = inputs.reshape(b * t).astype(jnp.int32)
    out = _make_gather(b * t)(idx, table)
    return out[:, :DIM].reshape(b, t, DIM)


# speedup vs baseline: 1.0011x; 1.0008x over previous
import functools

import jax
import jax.numpy as jnp
from jax import lax
from jax.experimental import pallas as pl
from jax.experimental.pallas import tpu as pltpu
from jax.experimental.pallas import tpu_sc as plsc

DIM = 32
OUT_LANES = 128
NUM_WORKERS = 32
CHUNK = 800
NBUF = 4


@functools.lru_cache(maxsize=None)
def _make_gather(total_rows: int):
    rows_per_worker = total_rows // NUM_WORKERS
    nchunks = rows_per_worker // CHUNK
    nblocks = nchunks // NBUF
    assert rows_per_worker % CHUNK == 0 and nchunks % NBUF == 0 and nchunks >= 4
    mesh = plsc.VectorSubcoreMesh(core_axis_name="c", subcore_axis_name="s")

    @functools.partial(
        pl.kernel,
        mesh=mesh,
        compiler_params=pltpu.CompilerParams(use_tc_tiling_on_sc=False),
        out_type=jax.ShapeDtypeStruct((total_rows, OUT_LANES), jnp.float32),
        scratch_types=[
            pltpu.VMEM((rows_per_worker,), jnp.int32),
            pltpu.VMEM((NBUF, CHUNK, DIM), jnp.float32),
            [pltpu.SemaphoreType.DMA] * NBUF,
            [pltpu.SemaphoreType.DMA] * NBUF,
        ],
    )
    def gather_kernel(idx_hbm, table_hbm, out_hbm, idx_v, rows_v, sem_g, sem_s):
        wid = lax.axis_index("s") * 2 + lax.axis_index("c")
        base = wid * rows_per_worker

        pltpu.sync_copy(idx_hbm.at[pl.ds(base, rows_per_worker)], idx_v)

        def start_gather(g, b):
            pltpu.async_copy(
                table_hbm.at[idx_v.at[pl.ds(g * CHUNK, CHUNK)]],
                rows_v.at[b],
                sem_g[b],
            )

        def wait_gather(b):
            pltpu.make_async_copy(
                table_hbm.at[idx_v.at[pl.ds(0, CHUNK)]], rows_v.at[b], sem_g[b]
            ).wait()

        def out_slice(g):
            return out_hbm.at[
                pl.ds(base + g * CHUNK, CHUNK), pl.ds(0, DIM)
            ]

        def start_store(g, b):
            pltpu.async_copy(rows_v.at[b], out_slice(g), sem_s[b])

        def wait_store(b):
            pltpu.make_async_copy(rows_v.at[b], out_slice(0), sem_s[b]).wait()

        start_gather(0, 0)
        start_gather(1, 1)

        def block(blk, carry):
            for b in range(NBUF):
                g = blk * NBUF + b
                wait_gather(b)
                nb = (b + 2) % NBUF

                @pl.when(g >= 2)
                def _():
                    wait_store(nb)

                @pl.when(g + 2 < nchunks)
                def _():
                    start_gather(g + 2, nb)

                start_store(g, b)
            return carry

        lax.fori_loop(0, nblocks, block, 0)

        wait_store((nchunks - 2) % NBUF)
        wait_store((nchunks - 1) % NBUF)

    return gather_kernel


@jax.jit
def kernel(inputs, table):
    b, t = inputs.shape
    idx = inputs.reshape(b * t).astype(jnp.int32)
    out = _make_gather(b * t)(idx, table)
    return out[:, :DIM].reshape(b, t, DIM)
